# baseline (speedup 1.0000x reference)
import dataclasses
import functools
import math

import jax
import jax.numpy as jnp
from jax import lax
from jax.experimental import pallas as pl
from jax.experimental.pallas import tpu as pltpu
from jax.experimental.pallas import tpu_sc as plsc

_DEPTH = 12
_LEAVES = 1 << _DEPTH
_NODES2 = 2 * ((1 << _DEPTH) - 1)
_NODES2_PAD = 8192
_B = 1_000_000
_NC, _NS, _L = 2, 16, 16
_NW = _NC * _NS
_CHUNK = 31_264
_LOG2X12 = jnp.float32(_DEPTH * math.log(2.0))
_LN2 = jnp.float32(0.6931471805599453)
_SQRT2 = jnp.float32(1.4142135623730951)


def _log16(x):
    bits = lax.bitcast_convert_type(x, jnp.int32)
    e = (bits >> 23) - 127
    m = lax.bitcast_convert_type(
        (bits & 0x007FFFFF) | 0x3F800000, jnp.float32)
    big = m > _SQRT2
    m = jnp.where(big, m * jnp.float32(0.5), m)
    e = jnp.where(big, e + 1, e).astype(jnp.float32)
    s = (m - jnp.float32(1.0)) / (m + jnp.float32(1.0))
    s2 = s * s
    ln_m = s * (jnp.float32(2.0)
                + s2 * (jnp.float32(2.0 / 3.0)
                        + s2 * (jnp.float32(2.0 / 5.0)
                                + s2 * jnp.float32(2.0 / 7.0))))
    return e * _LN2 + ln_m


def _gather_16(tab_v, z16):
    zc = jnp.minimum(jnp.maximum(z16, jnp.float32(0.0)), jnp.float32(1.0 - 1e-8))
    leaf = (zc * jnp.float32(_LEAVES)).astype(jnp.int32)
    leaf = jnp.minimum(leaf, _LEAVES - 1)
    return plsc.load_gather(tab_v, [leaf])


def _sc_body(th_hbm, z_hbm, out_hbm, lt_v, tab_v, z_v, o_v, sem):
    wid = lax.axis_index("s") * _NC + lax.axis_index("c")
    base = jnp.minimum(wid * _CHUNK, _B - _CHUNK)
    zcopy = pltpu.async_copy(z_hbm.at[pl.ds(base, _CHUNK)], z_v, sem)
    pltpu.sync_copy(th_hbm, lt_v.at[pl.ds(0, _NODES2)])

    lanes = lax.iota(jnp.int32, _L)

    @plsc.parallel_loop(0, _NODES2_PAD, step=_L, unroll=2)
    def _(i):
        lt_v[pl.ds(i, _L)] = _log16(lt_v[pl.ds(i, _L)] + jnp.float32(1e-20))

    @plsc.parallel_loop(0, _LEAVES, step=_L, unroll=2)
    def _(i):
        j = lanes + i
        acc = jnp.full((_L,), _LOG2X12, jnp.float32)
        for l in range(_DEPTH):
            idx = (j >> (11 - l)) + ((1 << (l + 1)) - 2)
            acc = acc + plsc.load_gather(lt_v, [idx])
        tab_v[pl.ds(i, _L)] = acc

    zcopy.wait()

    @plsc.parallel_loop(0, _CHUNK, step=_L, unroll=4)
    def _(i):
        o_v[pl.ds(i, _L)] = _gather_16(tab_v, z_v[pl.ds(i, _L)])

    pltpu.sync_copy(o_v, out_hbm.at[pl.ds(base, _CHUNK)])


_CP = pltpu.CompilerParams()
if "needs_layout_passes" in pltpu.CompilerParams.__dataclass_fields__:
    _CP = dataclasses.replace(_CP, needs_layout_passes=False)


@functools.partial(
    pl.kernel,
    mesh=plsc.VectorSubcoreMesh(core_axis_name="c", subcore_axis_name="s"),
    compiler_params=_CP,
    out_type=jax.ShapeDtypeStruct((_B,), jnp.float32),
    scratch_types=[
        pltpu.VMEM((_NODES2_PAD,), jnp.float32),
        pltpu.VMEM((_LEAVES,), jnp.float32),
        pltpu.VMEM((_CHUNK,), jnp.float32),
        pltpu.VMEM((_CHUNK,), jnp.float32),
        pltpu.SemaphoreType.DMA,
    ],
)
def _sc_kernel(th_hbm, z_hbm, out_hbm, lt_v, tab_v, z_v, o_v, sem):
    _sc_body(th_hbm, z_hbm, out_hbm, lt_v, tab_v, z_v, o_v, sem)


def kernel(z, theta):
    return _sc_kernel(jnp.reshape(theta, (-1,)), z)


# speedup vs baseline: 1.1325x; 1.1325x over previous
import dataclasses
import functools
import math

import jax
import jax.numpy as jnp
from jax import lax
from jax.experimental import pallas as pl
from jax.experimental.pallas import tpu as pltpu
from jax.experimental.pallas import tpu_sc as plsc

_DEPTH = 12
_LEAVES = 1 << _DEPTH
_NODES2 = 2 * ((1 << _DEPTH) - 1)
_NODES2_PAD = 8192
_B = 1_000_000
_NC, _NS, _L = 2, 16, 16
_NW = _NC * _NS
_CHUNK = 31_296
_NSUB = 4
_SUB = _CHUNK // _NSUB
_LOG2X12 = jnp.float32(_DEPTH * math.log(2.0))


def _log_body(x_ref, o_ref):
    o_ref[...] = jnp.log(x_ref[...] + 1e-20)


def _log_theta(theta):
    tf = jnp.reshape(theta, (-1,))
    tf = jnp.pad(tf, (0, _NODES2_PAD - _NODES2), constant_values=1.0)
    lt = pl.pallas_call(
        _log_body,
        out_shape=jax.ShapeDtypeStruct((_NODES2_PAD // 128, 128), jnp.float32),
    )(tf.reshape(_NODES2_PAD // 128, 128))
    return lt.reshape(_NODES2_PAD)


def _gather_16(tab_v, z16):
    zc = jnp.minimum(jnp.maximum(z16, jnp.float32(0.0)), jnp.float32(1.0 - 1e-8))
    leaf = (zc * jnp.float32(_LEAVES)).astype(jnp.int32)
    leaf = jnp.minimum(leaf, _LEAVES - 1)
    return plsc.load_gather(tab_v, [leaf])


def _sc_body(lt_hbm, z_hbm, out_hbm, lt_v, tab_v, tb_v, z_v, o_v, sems):
    zsems = sems[:_NSUB]
    osem = sems[_NSUB]
    wid = lax.axis_index("s") * _NC + lax.axis_index("c")
    base = jnp.minimum(wid * _CHUNK, _B - _CHUNK)

    zcopies = [
        pltpu.async_copy(
            z_hbm.at[pl.ds(base + s * _SUB, _SUB)],
            z_v.at[pl.ds(s * _SUB, _SUB)],
            zsems[s],
        )
        for s in range(_NSUB)
    ]
    pltpu.sync_copy(lt_hbm, lt_v)

    lanes = lax.iota(jnp.int32, _L)

    acc = jnp.full((_L,), _LOG2X12, jnp.float32)
    for l in range(4):
        idx = (lanes >> (3 - l)) + ((1 << (l + 1)) - 2)
        acc = acc + plsc.load_gather(lt_v, [idx])
    tab_v[pl.ds(0, _L)] = acc

    cur, nxt = tab_v, tb_v
    for l in range(4, _DEPTH):
        n = 1 << (l + 1)
        base_l = n - 2

        @plsc.parallel_loop(0, n, step=_L, unroll=min(4, n // _L))
        def _(i, cur=cur, nxt=nxt, base_l=base_l):
            k = lanes + i
            nxt[pl.ds(i, _L)] = (
                plsc.load_gather(cur, [k >> 1])
                + plsc.load_gather(lt_v, [base_l + k])
            )

        cur, nxt = nxt, cur

    ocopies = []
    for s in range(_NSUB):
        zcopies[s].wait()

        @plsc.parallel_loop(s * _SUB, (s + 1) * _SUB, step=_L, unroll=8)
        def _(i):
            o_v[pl.ds(i, _L)] = _gather_16(tab_v, z_v[pl.ds(i, _L)])

        ocopies.append(
            pltpu.async_copy(
                o_v.at[pl.ds(s * _SUB, _SUB)],
                out_hbm.at[pl.ds(base + s * _SUB, _SUB)],
                osem,
            )
        )

    for c in ocopies:
        c.wait()


_CP = pltpu.CompilerParams()
if "needs_layout_passes" in pltpu.CompilerParams.__dataclass_fields__:
    _CP = dataclasses.replace(_CP, needs_layout_passes=False)


@functools.partial(
    pl.kernel,
    mesh=plsc.VectorSubcoreMesh(core_axis_name="c", subcore_axis_name="s"),
    compiler_params=_CP,
    out_type=jax.ShapeDtypeStruct((_B,), jnp.float32),
    scratch_types=[
        pltpu.VMEM((_NODES2_PAD,), jnp.float32),
        pltpu.VMEM((_LEAVES,), jnp.float32),
        pltpu.VMEM((_LEAVES // 2,), jnp.float32),
        pltpu.VMEM((_CHUNK,), jnp.float32),
        pltpu.VMEM((_CHUNK,), jnp.float32),
        [pltpu.SemaphoreType.DMA] * (_NSUB + 1),
    ],
)
def _sc_kernel(lt_hbm, z_hbm, out_hbm, lt_v, tab_v, tb_v, z_v, o_v, sems):
    _sc_body(lt_hbm, z_hbm, out_hbm, lt_v, tab_v, tb_v, z_v, o_v, sems)


def kernel(z, theta):
    return _sc_kernel(_log_theta(theta), z)
